# baseline (speedup 1.0000x reference)
import jax
import jax.numpy as jnp
from jax import lax
from jax.experimental import pallas as pl
from jax.experimental.pallas import tpu as pltpu
from jax.experimental.pallas import tpu_sc as plsc

_VOCAB = 1000000
_D = 64
_BATCH = 16384
_HIST = 20
_N = _BATCH * _HIST

_NC = 2
_NS = 16
_NW = _NC * _NS
_CHUNK = 128
_PER_W = _N // _NW
_NCH = _PER_W // _CHUNK

_mesh = plsc.VectorSubcoreMesh(
    core_axis_name="c", subcore_axis_name="s",
    num_cores=_NC, num_subcores=_NS,
)


def _body(idx_hbm, tab_hbm, out_hbm, idx_v, rows_v, sem):
    wid = lax.axis_index("s") * _NC + lax.axis_index("c")
    pltpu.sync_copy(idx_hbm.at[wid], idx_v)

    def step(j, carry):
        cid = wid * _NCH + j
        pltpu.async_copy(tab_hbm.at[idx_v.at[j]], rows_v, sem).wait()
        pltpu.sync_copy(rows_v, out_hbm.at[cid])
        return carry

    lax.fori_loop(0, _NCH, step, 0)


_gather = pl.kernel(
    _body,
    out_type=jax.ShapeDtypeStruct((_NW * _NCH, _CHUNK, _D), jnp.float32),
    mesh=_mesh,
    scratch_types=[
        pltpu.VMEM((_NCH, _CHUNK), jnp.int32),
        pltpu.VMEM((_CHUNK, _D), jnp.float32),
        pltpu.SemaphoreType.DMA,
    ],
    compiler_params=pltpu.CompilerParams(use_tc_tiling_on_sc=False),
)


def kernel(inputs, w, b):
    idx = inputs.astype(jnp.int32).reshape(_NW, _NCH, _CHUNK)
    out = _gather(idx, w)
    return out.reshape(_BATCH, _HIST, _D)


# speedup vs baseline: 1.0668x; 1.0668x over previous
import jax
import jax.numpy as jnp
from jax import lax
from jax.experimental import pallas as pl
from jax.experimental.pallas import tpu as pltpu
from jax.experimental.pallas import tpu_sc as plsc

_VOCAB = 1000000
_D = 64
_BATCH = 16384
_HIST = 20
_N = _BATCH * _HIST

_NC = 2
_NS = 16
_NW = _NC * _NS
_CHUNK = 128
_PER_W = _N // _NW
_NCH = _PER_W // _CHUNK
_G = 4
_ROWS = _G * _CHUNK
_NG = _NCH // _G
_NP = _NG // 2

_mesh = plsc.VectorSubcoreMesh(
    core_axis_name="c", subcore_axis_name="s",
    num_cores=_NC, num_subcores=_NS,
)


def _body(idx_hbm, tab_hbm, out_hbm, idx_v, bufs, gsems, osems):
    wid = lax.axis_index("s") * _NC + lax.axis_index("c")
    pltpu.sync_copy(idx_hbm.at[wid], idx_v)

    def fire_gathers(g, p):
        for k in range(_G):
            pltpu.async_copy(
                tab_hbm.at[idx_v.at[g * _G + k]],
                bufs.at[p, pl.ds(k * _CHUNK, _CHUNK)],
                gsems.at[p],
            )

    def drain_gathers(p):
        pltpu.make_async_copy(out_hbm.at[0], bufs.at[p], gsems.at[p]).wait()

    def fire_out(g, p):
        pltpu.async_copy(bufs.at[p], out_hbm.at[wid * _NG + g], osems.at[p])

    def wait_out(g, p):
        pltpu.make_async_copy(bufs.at[p], out_hbm.at[wid * _NG + g],
                              osems.at[p]).wait()

    fire_gathers(0, 0)
    fire_gathers(1, 1)

    def step(t, carry):
        for p in range(2):
            g = 2 * t + p
            drain_gathers(p)
            fire_out(g, p)
            wait_out(g, p)

            @pl.when(t < _NP - 1)
            def _():
                fire_gathers(g + 2, p)

        return carry

    lax.fori_loop(0, _NP, step, 0)


_gather = pl.kernel(
    _body,
    out_type=jax.ShapeDtypeStruct((_NW * _NG, _ROWS, _D), jnp.float32),
    mesh=_mesh,
    scratch_types=[
        pltpu.VMEM((_NCH, _CHUNK), jnp.int32),
        pltpu.VMEM((2, _ROWS, _D), jnp.float32),
        pltpu.SemaphoreType.DMA((2,)),
        pltpu.SemaphoreType.DMA((2,)),
    ],
    compiler_params=pltpu.CompilerParams(use_tc_tiling_on_sc=False),
)


def kernel(inputs, w, b):
    idx = inputs.astype(jnp.int32).reshape(_NW, _NCH, _CHUNK)
    out = _gather(idx, w)
    return out.reshape(_BATCH, _HIST, _D)
